# baseline (speedup 1.0000x reference)
import jax
import jax.numpy as jnp
from jax import lax
from jax.experimental import pallas as pl

N_DIM = 512
T = 25
NDIA = 32
DB = 16
RB = DB * T
GRID = NDIA // DB
NUM_L = 3
NUM_K = 4
ROWS = NDIA * T
NE_D = T + NUM_L
NI_D = 6 * T


def _body(l_ref, a_ref, v_ref, tab_ref, emb_ref,
          w1_ref, b1_ref, attrs_ref, hw_ref, hb_ref, gw_ref, gb_ref, out_ref):
    f32 = jnp.float32
    i32 = jnp.int32
    k = pl.program_id(0)
    SR = 3 * RB

    def dot(x, w):
        return jax.lax.dot(x, w, preferred_element_type=f32)

    def tile3(x):
        return jnp.concatenate([x, x, x], axis=0)

    def lsum(x):
        return jnp.sum(x, axis=1, keepdims=True)

    r_i = lax.broadcasted_iota(i32, (RB, 1), 0)
    t_i = r_i % T
    d_i = r_i // T + DB * k

    Ad = (lax.broadcasted_iota(i32, (RB, NDIA), 1) == d_i).astype(f32)
    At = (lax.broadcasted_iota(i32, (RB, T), 1) == t_i).astype(f32)

    rowT = dot(Ad, tab_ref[...])

    def pick(lo, width, idx):
        sub = rowT[:, lo:lo + width]
        c = lax.broadcasted_iota(i32, (RB, width), 1)
        return lsum(sub * (c == idx))

    ewb_l = pick(0, T, t_i)
    ewb_a = pick(T, T, t_i)
    ewb_v = pick(2 * T, T, t_i)
    ewt_l = pick(3 * T, 3 * T, 3 * t_i)
    ewt_a = pick(3 * T, 3 * T, 3 * t_i + 1)
    ewt_v = pick(3 * T, 3 * T, 3 * t_i + 2)
    OH = 6 * T
    wbig_l = rowT[:, OH:OH + 1]
    wbig_a = rowT[:, OH + 1:OH + 2]
    wbig_v = rowT[:, OH + 2:OH + 3]
    wtri = pick(OH + NUM_L, T, t_i)
    OQ = OH + NE_D
    c50 = lax.broadcasted_iota(i32, (RB, 2 * T), 1)
    qdiff = lsum(rowT[:, OQ:OQ + 2 * T]
                 * ((c50 == 2 * t_i).astype(f32)
                    - (c50 == 2 * t_i + 1).astype(f32)))
    sel = (qdiff >= 0).astype(f32)
    dlen = rowT[:, OQ + 2 * T:OQ + 2 * T + 1]
    mask = (t_i.astype(f32) < dlen).astype(f32)
    tmask = (t_i < (T - NUM_L)).astype(f32)
    emb = emb_ref[...]
    embsel = emb[1:2, :] + sel * (emb[0:1, :] - emb[1:2, :])

    F = jnp.concatenate([l_ref[...] * mask + embsel,
                         a_ref[...] * mask,
                         v_ref[...] * mask], axis=0)
    x1 = dot(F, w1_ref[...]) + b1_ref[...]

    g_io = lax.broadcasted_iota(i32, (3 * DB, SR), 0)
    r_io = lax.broadcasted_iota(i32, (3 * DB, SR), 1)
    U = (r_io // T == g_io).astype(f32)
    g_io2 = lax.broadcasted_iota(i32, (SR, 3 * DB), 1)
    r_io2 = lax.broadcasted_iota(i32, (SR, 3 * DB), 0)
    UT = (r_io2 // T == g_io2).astype(f32)

    inv_tri = 1.0 / 3.0
    dinv_g = 1.0 / (T - 1 + 2)
    UT_h = UT * (1.0 / T)
    UT_g = UT * dinv_g

    def gsum_h(x):
        return dot(UT_h, dot(U, x))

    def gsum_g(x):
        return dot(UT_g, dot(U, x))

    ewb_s = jnp.concatenate([ewb_l, ewb_a, ewb_v], axis=0)
    wbig_s = jnp.concatenate([wbig_l, wbig_a, wbig_v], axis=0)
    wtri_s = tile3(wtri)
    dd = wbig_s + wtri_s
    dinv_s = jnp.where(dd > 0, 1.0 / dd, 0.0)
    db_s = wbig_s * dinv_s
    ct_s = wtri_s * dinv_s

    H = x1
    attrs = attrs_ref[...]
    for li in range(NUM_L):
        W = hw_ref[li]
        b = hb_ref[li:li + 1, :]
        ew12 = dot(attrs, W)
        ew1 = ew12[0:1, :]
        ew2 = ew12[1:2, :]
        xw = dot(H, W)
        ewtri = ew2 + tmask * (ew1 - ew2)
        mtri = (xw[0:RB] * ewt_l + xw[RB:2 * RB] * ewt_a
                + xw[2 * RB:] * ewt_v) * inv_tri + ewtri
        H = db_s * gsum_h(xw * ewb_s) + (db_s * ew1 + b) + ct_s * tile3(mtri)

    G = x1
    for kk in range(NUM_K):
        W = gw_ref[kk]
        b = gb_ref[kk:kk + 1, :]
        xp = dot(G, W)
        T3d = (xp[0:RB] + xp[RB:2 * RB] + xp[2 * RB:]) * dinv_g
        G = G + gsum_g(xp) + tile3(T3d) - (2.0 * dinv_g) * xp + b

    D = N_DIM
    for g in range(3):
        sl = slice(g * RB, (g + 1) * RB)
        out_ref[:, (3 * g) * D:(3 * g + 1) * D] = F[sl]
        out_ref[:, (3 * g + 1) * D:(3 * g + 2) * D] = H[sl]
        out_ref[:, (3 * g + 2) * D:(3 * g + 3) * D] = G[sl]


def kernel(a, v, l, dia_len, qmask, epoch, speaker_emb, fc1_W, fc1_b,
           hyperedge_weight, EW_weight, hyperedge_attr1, hyperedge_attr2,
           hconv_W, hconv_b, gconv_W, gconv_b):
    f32 = jnp.float32
    num_edges = NDIA * NE_D
    nnz = NDIA * NI_D

    tab = jnp.concatenate([
        EW_weight[:nnz].reshape(NDIA, NI_D),
        hyperedge_weight[:num_edges].reshape(NDIA, NE_D),
        qmask.transpose(1, 0, 2).reshape(NDIA, 2 * T),
        dia_len.astype(f32).reshape(NDIA, 1),
        jnp.zeros((NDIA, 256 - NI_D - NE_D - 2 * T - 1), f32),
    ], axis=1)
    attrs = jnp.stack([hyperedge_attr1, hyperedge_attr2], axis=0)
    b1 = fc1_b.reshape(1, N_DIM)

    row_spec = pl.BlockSpec((RB, N_DIM), lambda k: (k, 0))
    full2 = lambda arr: pl.BlockSpec(arr.shape, lambda k: (0,) * arr.ndim)

    out = pl.pallas_call(
        _body,
        grid=(GRID,),
        in_specs=[
            row_spec, row_spec, row_spec,
            full2(tab),
            full2(speaker_emb), full2(fc1_W), full2(b1), full2(attrs),
            full2(hconv_W), full2(hconv_b), full2(gconv_W), full2(gconv_b),
        ],
        out_specs=pl.BlockSpec((RB, 9 * N_DIM), lambda k: (k, 0)),
        out_shape=jax.ShapeDtypeStruct((ROWS, 9 * N_DIM), f32),
    )(l, a, v, tab, speaker_emb, fc1_W, b1, attrs,
      hconv_W, hconv_b, gconv_W, gconv_b)
    return out


# speedup vs baseline: 1.0929x; 1.0929x over previous
import jax
import jax.numpy as jnp
from jax import lax
from jax.experimental import pallas as pl

N_DIM = 512
T = 25
NDIA = 32
DB = 16
RB = DB * T
GRID = NDIA // DB
NUM_L = 3
NUM_K = 4
ROWS = NDIA * T
NE_D = T + NUM_L
NI_D = 6 * T


def _body(l_ref, a_ref, v_ref, dlen_ref, qm_ref, e2_ref, hw2_ref, emb_ref,
          w1_ref, b1_ref, attrs_ref, hw_ref, hb_ref, gw_ref, gb_ref, out_ref):
    f32 = jnp.float32
    i32 = jnp.int32
    k = pl.program_id(0)
    SR = 3 * RB

    def dot(x, w):
        return jax.lax.dot(x, w, preferred_element_type=f32)

    def tile3(x):
        return jnp.concatenate([x, x, x], axis=0)

    def lsum(x):
        return jnp.sum(x, axis=1, keepdims=True)

    r_i = lax.broadcasted_iota(i32, (RB, 1), 0)
    t_i = r_i % T
    d_i = r_i // T + DB * k

    Ad = (lax.broadcasted_iota(i32, (RB, NDIA), 1) == d_i).astype(f32)
    At = (lax.broadcasted_iota(i32, (RB, T), 1) == t_i).astype(f32)

    dlen = dot(Ad, dlen_ref[...])
    mask = (t_i.astype(f32) < dlen).astype(f32)
    tmask = (t_i < (T - NUM_L)).astype(f32)

    j_io = lax.broadcasted_iota(i32, (2 * NDIA, NDIA), 0)
    d_io = lax.broadcasted_iota(i32, (2 * NDIA, NDIA), 1)
    W64 = ((j_io == 2 * d_io).astype(f32)
           - (j_io == 2 * d_io + 1).astype(f32))
    Dq = dot(qm_ref[...], W64)
    qdiff = lsum(dot(At, Dq) * Ad)
    sel = (qdiff >= 0).astype(f32)
    emb = emb_ref[...]
    embsel = emb[1:2, :] + sel * (emb[0:1, :] - emb[1:2, :])

    rowE = dot(Ad, e2_ref[...])
    c150 = lax.broadcasted_iota(i32, (RB, NI_D), 1)
    ewb_l = lsum(rowE * (c150 == t_i))
    ewb_a = lsum(rowE * (c150 == T + t_i))
    ewb_v = lsum(rowE * (c150 == 2 * T + t_i))
    ewt_l = lsum(rowE * (c150 == 3 * T + 3 * t_i))
    ewt_a = lsum(rowE * (c150 == 3 * T + 3 * t_i + 1))
    ewt_v = lsum(rowE * (c150 == 3 * T + 3 * t_i + 2))

    rowH = dot(Ad, hw2_ref[...])
    c28 = lax.broadcasted_iota(i32, (RB, NE_D), 1)
    wbig_l = rowH[:, 0:1]
    wbig_a = rowH[:, 1:2]
    wbig_v = rowH[:, 2:3]
    wtri = lsum(rowH * (c28 == NUM_L + t_i))

    F = jnp.concatenate([l_ref[...] * mask + embsel,
                         a_ref[...] * mask,
                         v_ref[...] * mask], axis=0)
    x1 = dot(F, w1_ref[...]) + b1_ref[...]

    g_io = lax.broadcasted_iota(i32, (3 * DB, SR), 0)
    r_io = lax.broadcasted_iota(i32, (3 * DB, SR), 1)
    U = (r_io // T == g_io).astype(f32)
    g_io2 = lax.broadcasted_iota(i32, (SR, 3 * DB), 1)
    r_io2 = lax.broadcasted_iota(i32, (SR, 3 * DB), 0)
    UT = (r_io2 // T == g_io2).astype(f32)

    inv_tri = 1.0 / 3.0
    dinv_g = 1.0 / (T - 1 + 2)
    UT_h = UT * (1.0 / T)
    UT_g = UT * dinv_g

    def gsum_h(x):
        return dot(UT_h, dot(U, x))

    def gsum_g(x):
        return dot(UT_g, dot(U, x))

    ewb_s = jnp.concatenate([ewb_l, ewb_a, ewb_v], axis=0)
    wbig_s = jnp.concatenate([wbig_l, wbig_a, wbig_v], axis=0)
    wtri_s = tile3(wtri)
    dd = wbig_s + wtri_s
    dinv_s = jnp.where(dd > 0, 1.0 / dd, 0.0)
    db_s = wbig_s * dinv_s
    ct_s = wtri_s * dinv_s

    H = x1
    attrs = attrs_ref[...]
    for li in range(NUM_L):
        W = hw_ref[li]
        b = hb_ref[li:li + 1, :]
        ew12 = dot(attrs, W)
        ew1 = ew12[0:1, :]
        ew2 = ew12[1:2, :]
        xw = dot(H, W)
        ewtri = ew2 + tmask * (ew1 - ew2)
        mtri = (xw[0:RB] * ewt_l + xw[RB:2 * RB] * ewt_a
                + xw[2 * RB:] * ewt_v) * inv_tri + ewtri
        H = db_s * gsum_h(xw * ewb_s) + (db_s * ew1 + b) + ct_s * tile3(mtri)

    G = x1
    for kk in range(NUM_K):
        W = gw_ref[kk]
        b = gb_ref[kk:kk + 1, :]
        xp = dot(G, W)
        T3d = (xp[0:RB] + xp[RB:2 * RB] + xp[2 * RB:]) * dinv_g
        G = G + gsum_g(xp) + tile3(T3d) - (2.0 * dinv_g) * xp + b

    D = N_DIM
    for g in range(3):
        sl = slice(g * RB, (g + 1) * RB)
        out_ref[:, (3 * g) * D:(3 * g + 1) * D] = F[sl]
        out_ref[:, (3 * g + 1) * D:(3 * g + 2) * D] = H[sl]
        out_ref[:, (3 * g + 2) * D:(3 * g + 3) * D] = G[sl]


def kernel(a, v, l, dia_len, qmask, epoch, speaker_emb, fc1_W, fc1_b,
           hyperedge_weight, EW_weight, hyperedge_attr1, hyperedge_attr2,
           hconv_W, hconv_b, gconv_W, gconv_b):
    f32 = jnp.float32
    num_edges = NDIA * NE_D
    nnz = NDIA * NI_D

    dlen32 = dia_len.reshape(NDIA, 1).astype(f32)
    qmflat = qmask.astype(f32).reshape(T, 2 * NDIA)
    E2 = EW_weight.astype(f32)[:nnz].reshape(NDIA, NI_D)
    hw2 = hyperedge_weight.astype(f32)[:num_edges].reshape(NDIA, NE_D)
    attrs = jnp.stack([hyperedge_attr1, hyperedge_attr2], axis=0).astype(f32)
    b1 = fc1_b.reshape(1, N_DIM).astype(f32)

    row_spec = pl.BlockSpec((RB, N_DIM), lambda k: (k, 0))
    full2 = lambda arr: pl.BlockSpec(arr.shape, lambda k: (0,) * arr.ndim)

    out = pl.pallas_call(
        _body,
        grid=(GRID,),
        in_specs=[
            row_spec, row_spec, row_spec,
            full2(dlen32), full2(qmflat), full2(E2), full2(hw2),
            full2(speaker_emb), full2(fc1_W), full2(b1), full2(attrs),
            full2(hconv_W), full2(hconv_b), full2(gconv_W), full2(gconv_b),
        ],
        out_specs=pl.BlockSpec((RB, 9 * N_DIM), lambda k: (k, 0)),
        out_shape=jax.ShapeDtypeStruct((ROWS, 9 * N_DIM), f32),
    )(l.astype(f32), a.astype(f32), v.astype(f32), dlen32, qmflat, E2, hw2,
      speaker_emb.astype(f32), fc1_W.astype(f32), b1, attrs,
      hconv_W.astype(f32), hconv_b.astype(f32),
      gconv_W.astype(f32), gconv_b.astype(f32))
    return out
